# baseline (speedup 1.0000x reference)
import functools

import jax
import jax.numpy as jnp
from jax import lax
from jax.experimental import pallas as pl
from jax.experimental.pallas import tpu as pltpu
from jax.experimental.pallas import tpu_sc as plsc

_D = 32
_T = 50
_J = 5
_IB = 128
_DT = _D // 8


@functools.cache
def _make_lookup(NT: int, V: int):
    info = plsc.get_sparse_core_info()
    nc, ns = info.num_cores, info.num_subcores
    nw = nc * ns
    t_per_w = NT // nw
    nib = t_per_w // _IB
    nstrip = nib * (_T // _J)
    assert NT % nw == 0 and t_per_w % _IB == 0 and _T % _J == 0
    assert nstrip % 2 == 0

    mesh = plsc.VectorSubcoreMesh(core_axis_name="c", subcore_axis_name="s")

    @functools.partial(
        pl.kernel,
        mesh=mesh,
        compiler_params=pltpu.CompilerParams(
            use_tc_tiling_on_sc=False, needs_layout_passes=False
        ),
        out_type=jax.ShapeDtypeStruct((_T, _DT, NT // _IB, 8, _IB), jnp.float32),
        scratch_types=[
            pltpu.VMEM((_J, _IB), jnp.int32),
            pltpu.VMEM((_J, _IB), jnp.int32),
            pltpu.VMEM((_J, _IB, _D), jnp.float32),
            pltpu.VMEM((_J, _IB, _D), jnp.float32),
            pltpu.VMEM((_J, _DT, 8, _IB), jnp.float32),
            pltpu.VMEM((_J, _DT, 8, _IB), jnp.float32),
            pltpu.SemaphoreType.DMA,
            pltpu.SemaphoreType.DMA,
            pltpu.SemaphoreType.DMA,
            pltpu.SemaphoreType.DMA,
            pltpu.SemaphoreType.DMA,
            pltpu.SemaphoreType.DMA,
        ],
    )
    def lookup(table_hbm, idxT_hbm, out_hbm,
               idx0, idx1, rows0, rows1, tiles0, tiles1,
               si0, si1, sg0, sg1, so0, so1):
        wid = lax.axis_index("s") * nc + lax.axis_index("c")
        ib0 = wid * nib
        idx_b = (idx0, idx1)
        rows_b = (rows0, rows1)
        tiles_b = (tiles0, tiles1)
        si = (si0, si1)
        sg = (sg0, sg1)
        so = (so0, so1)
        nsj = _T // _J

        def strip_coords(u):
            ib = ib0 + u // nsj
            jj0 = (u % nsj) * _J
            return ib, jj0

        def idx_copy(u, p):
            ib, jj0 = strip_coords(u)
            i0 = pl.multiple_of(ib * _IB, 128)
            return pltpu.make_async_copy(
                idxT_hbm.at[pl.ds(jj0, _J), pl.ds(i0, _IB)], idx_b[p], si[p]
            )

        def gathers(u, p, wait):
            for j in range(_J):
                cp = pltpu.make_async_copy(
                    table_hbm.at[idx_b[p].at[j]], rows_b[p].at[j], sg[p]
                )
                if wait:
                    cp.wait()
                else:
                    cp.start()

        def tile_stores(u, p, wait):
            ib, jj0 = strip_coords(u)
            for j in range(_J):
                for dt in range(_DT):
                    cp = pltpu.make_async_copy(
                        tiles_b[p].at[j, dt], out_hbm.at[jj0 + j, dt, ib], so[p]
                    )
                    if wait:
                        cp.wait()
                    else:
                        cp.start()

        def build_tiles(p):
            lanes = lax.iota(jnp.int32, 16)
            zeros = jnp.zeros((16,), jnp.int32)

            @pl.loop(0, _J)
            def _j(j):
                jv = zeros + j

                @pl.loop(0, _DT)
                def _dt(dt):
                    for dr in range(8):
                        dvec = zeros + (dt * 8 + dr)
                        for ig in range(_IB // 16):
                            ivec = lanes + (ig * 16)
                            vals = plsc.load_gather(
                                rows_b[p], [jv, ivec, dvec]
                            )
                            tiles_b[p][j, dt, dr, pl.ds(ig * 16, 16)] = vals

        idx_copy(0, 0).start()

        def stage(u, p):
            q = 1 - p
            idx_copy(u, p).wait()
            gathers(u, p, wait=False)

            @pl.when(u >= 1)
            def _():
                gathers(u - 1, q, wait=True)

                @pl.when(u >= 3)
                def _():
                    tile_stores(u - 3, q, wait=True)

                build_tiles(q)
                tile_stores(u - 1, q, wait=False)

            @pl.when(u + 1 < nstrip)
            def _():
                idx_copy(u + 1, q).start()

        @pl.loop(0, nstrip, step=2)
        def _pair(u0):
            stage(u0, 0)
            stage(u0 + 1, 1)

        last = nstrip - 1
        gathers(last, 1, wait=True)
        tile_stores(last - 2, 1, wait=True)
        build_tiles(1)
        tile_stores(last, 1, wait=False)
        tile_stores(last - 1, 0, wait=True)
        tile_stores(last, 1, wait=True)

    return lookup


def kernel(indices, weight):
    NT, T = indices.shape
    out5 = _make_lookup(NT, weight.shape[0])(
        weight, indices.T.astype(jnp.int32)
    )
    return out5.transpose(2, 4, 0, 1, 3).reshape(NT, T, _D)


# speedup vs baseline: 1.3130x; 1.3130x over previous
import functools

import jax
import jax.numpy as jnp
from jax import lax
from jax.experimental import pallas as pl
from jax.experimental.pallas import tpu as pltpu
from jax.experimental.pallas import tpu_sc as plsc

_D = 32
_T = 50
_J = 5
_IB = 128
_DT = _D // 8


@functools.cache
def _make_lookup(NT: int, V: int):
    info = plsc.get_sparse_core_info()
    nc, ns = info.num_cores, info.num_subcores
    nw = nc * ns
    t_per_w = NT // nw
    nib = t_per_w // _IB
    nstrip = nib * (_T // _J)
    assert NT % nw == 0 and t_per_w % _IB == 0 and _T % _J == 0
    assert nstrip % 2 == 0

    mesh = plsc.VectorSubcoreMesh(core_axis_name="c", subcore_axis_name="s")

    @functools.partial(
        pl.kernel,
        mesh=mesh,
        compiler_params=pltpu.CompilerParams(
            use_tc_tiling_on_sc=False, needs_layout_passes=False
        ),
        out_type=jax.ShapeDtypeStruct((_T, _DT, NT // _IB, 8, _IB), jnp.float32),
        scratch_types=[
            pltpu.VMEM((_J, _IB), jnp.int32),
            pltpu.VMEM((_J, _IB), jnp.int32),
            pltpu.VMEM((_J, _IB, _D), jnp.float32),
            pltpu.VMEM((_J, _IB, _D), jnp.float32),
            pltpu.VMEM((_J, _DT, 8, _IB), jnp.float32),
            pltpu.VMEM((_J, _DT, 8, _IB), jnp.float32),
            pltpu.SemaphoreType.DMA,
            pltpu.SemaphoreType.DMA,
            pltpu.SemaphoreType.DMA,
            pltpu.SemaphoreType.DMA,
            pltpu.SemaphoreType.DMA,
            pltpu.SemaphoreType.DMA,
        ],
    )
    def lookup(table_hbm, idxT_hbm, out_hbm,
               idx0, idx1, rows0, rows1, tiles0, tiles1,
               si0, si1, sg0, sg1, so0, so1):
        wid = lax.axis_index("s") * nc + lax.axis_index("c")
        ib0 = wid * nib
        idx_b = (idx0, idx1)
        rows_b = (rows0, rows1)
        tiles_b = (tiles0, tiles1)
        si = (si0, si1)
        sg = (sg0, sg1)
        so = (so0, so1)
        nsj = _T // _J

        def strip_coords(u):
            ib = ib0 + u // nsj
            jj0 = (u % nsj) * _J
            return ib, jj0

        def idx_copy(u, p):
            ib, jj0 = strip_coords(u)
            i0 = pl.multiple_of(ib * _IB, 128)
            return pltpu.make_async_copy(
                idxT_hbm.at[pl.ds(jj0, _J), pl.ds(i0, _IB)], idx_b[p], si[p]
            )

        def gathers(u, p, wait):
            for j in range(_J):
                cp = pltpu.make_async_copy(
                    table_hbm.at[idx_b[p].at[j]], rows_b[p].at[j], sg[p]
                )
                if wait:
                    cp.wait()
                else:
                    cp.start()

        def tile_stores(u, p, wait):
            ib, jj0 = strip_coords(u)
            for j in range(_J):
                for dt in range(_DT):
                    cp = pltpu.make_async_copy(
                        tiles_b[p].at[j, dt], out_hbm.at[jj0 + j, dt, ib], so[p]
                    )
                    if wait:
                        cp.wait()
                    else:
                        cp.start()

        def build_tiles(p):
            lanes = lax.iota(jnp.int32, 16)
            zeros = jnp.zeros((16,), jnp.int32)

            @pl.loop(0, _J)
            def _j(j):
                jv = zeros + j
                for dt in range(_DT):
                    dvecs = [zeros + (dt * 8 + dr) for dr in range(8)]
                    for ig in range(_IB // 16):
                        ivec = lanes + (ig * 16)
                        vals = [
                            plsc.load_gather(rows_b[p], [jv, ivec, dvecs[dr]])
                            for dr in range(8)
                        ]
                        for dr in range(8):
                            tiles_b[p][j, dt, dr, pl.ds(ig * 16, 16)] = vals[dr]

        idx_copy(0, 0).start()

        def stage(u, p):
            q = 1 - p
            idx_copy(u, p).wait()
            gathers(u, p, wait=False)

            @pl.when(u >= 1)
            def _():
                gathers(u - 1, q, wait=True)

                @pl.when(u >= 3)
                def _():
                    tile_stores(u - 3, q, wait=True)

                build_tiles(q)
                tile_stores(u - 1, q, wait=False)

            @pl.when(u + 1 < nstrip)
            def _():
                idx_copy(u + 1, q).start()

        @pl.loop(0, nstrip, step=2)
        def _pair(u0):
            stage(u0, 0)
            stage(u0 + 1, 1)

        last = nstrip - 1
        gathers(last, 1, wait=True)
        tile_stores(last - 2, 1, wait=True)
        build_tiles(1)
        tile_stores(last, 1, wait=False)
        tile_stores(last - 1, 0, wait=True)
        tile_stores(last, 1, wait=True)

    return lookup


def kernel(indices, weight):
    NT, T = indices.shape
    out5 = _make_lookup(NT, weight.shape[0])(
        weight, indices.T.astype(jnp.int32)
    )
    return out5.transpose(2, 4, 0, 1, 3).reshape(NT, T, _D)


# speedup vs baseline: 1.3265x; 1.0103x over previous
import functools

import jax
import jax.numpy as jnp
from jax import lax
from jax.experimental import pallas as pl
from jax.experimental.pallas import tpu as pltpu
from jax.experimental.pallas import tpu_sc as plsc

_D = 32
_T = 50
_J = 5
_IB = 128
_DT = _D // 8


@functools.cache
def _make_lookup(NT: int, V: int):
    info = plsc.get_sparse_core_info()
    nc, ns = info.num_cores, info.num_subcores
    nw = nc * ns
    t_per_w = NT // nw
    nib = t_per_w // _IB
    nstrip = nib * (_T // _J)
    assert NT % nw == 0 and t_per_w % _IB == 0 and _T % _J == 0
    assert nstrip % 2 == 0

    mesh = plsc.VectorSubcoreMesh(core_axis_name="c", subcore_axis_name="s")

    @functools.partial(
        pl.kernel,
        mesh=mesh,
        compiler_params=pltpu.CompilerParams(
            use_tc_tiling_on_sc=False, needs_layout_passes=False
        ),
        out_type=jax.ShapeDtypeStruct((_T, _DT, NT // _IB, 8, _IB), jnp.float32),
        scratch_types=[
            pltpu.VMEM((_J, _IB), jnp.int32),
            pltpu.VMEM((_J, _IB), jnp.int32),
            pltpu.VMEM((_J, _IB, _D), jnp.float32),
            pltpu.VMEM((_J, _IB, _D), jnp.float32),
            pltpu.VMEM((_J, _DT, 8, _IB), jnp.float32),
            pltpu.VMEM((_J, _DT, 8, _IB), jnp.float32),
            pltpu.SemaphoreType.DMA,
            pltpu.SemaphoreType.DMA,
            pltpu.SemaphoreType.DMA,
            pltpu.SemaphoreType.DMA,
            pltpu.SemaphoreType.DMA,
            pltpu.SemaphoreType.DMA,
        ],
    )
    def lookup(table_hbm, idxT_hbm, out_hbm,
               idx0, idx1, rows0, rows1, tiles0, tiles1,
               si0, si1, sg0, sg1, so0, so1):
        wid = lax.axis_index("s") * nc + lax.axis_index("c")
        ib0 = wid * nib
        idx_b = (idx0, idx1)
        rows_b = (rows0, rows1)
        tiles_b = (tiles0, tiles1)
        si = (si0, si1)
        sg = (sg0, sg1)
        so = (so0, so1)
        nsj = _T // _J

        def strip_coords(u):
            ib = ib0 + u // nsj
            jj0 = (u % nsj) * _J
            return ib, jj0

        def idx_copy(u, p):
            ib, jj0 = strip_coords(u)
            i0 = pl.multiple_of(ib * _IB, 128)
            return pltpu.make_async_copy(
                idxT_hbm.at[pl.ds(jj0, _J), pl.ds(i0, _IB)], idx_b[p], si[p]
            )

        def gathers(u, p, wait):
            for j in range(_J):
                cp = pltpu.make_async_copy(
                    table_hbm.at[idx_b[p].at[j]], rows_b[p].at[j], sg[p]
                )
                if wait:
                    cp.wait()
                else:
                    cp.start()

        def tile_stores(u, p, wait):
            ib, jj0 = strip_coords(u)
            for j in range(_J):
                for dt in range(_DT):
                    cp = pltpu.make_async_copy(
                        tiles_b[p].at[j, dt], out_hbm.at[jj0 + j, dt, ib], so[p]
                    )
                    if wait:
                        cp.wait()
                    else:
                        cp.start()

        def build_tiles(p):
            lanes = lax.iota(jnp.int32, 16)
            zeros = jnp.zeros((16,), jnp.int32)

            @pl.loop(0, _J)
            def _j(j):
                jv = zeros + j
                for dt in range(_DT):
                    dvecs = [zeros + (dt * 8 + dr) for dr in range(8)]
                    for ig2 in range(_IB // 32):
                        ivecs = [lanes + (ig2 * 32 + h * 16) for h in range(2)]
                        vals = [
                            plsc.load_gather(rows_b[p], [jv, ivecs[h], dvecs[dr]])
                            for h in range(2)
                            for dr in range(8)
                        ]
                        for h in range(2):
                            for dr in range(8):
                                tiles_b[p][
                                    j, dt, dr, pl.ds(ig2 * 32 + h * 16, 16)
                                ] = vals[h * 8 + dr]

        idx_copy(0, 0).start()

        def stage(u, p):
            q = 1 - p
            idx_copy(u, p).wait()
            gathers(u, p, wait=False)

            @pl.when(u >= 1)
            def _():
                gathers(u - 1, q, wait=True)

                @pl.when(u >= 3)
                def _():
                    tile_stores(u - 3, q, wait=True)

                build_tiles(q)
                tile_stores(u - 1, q, wait=False)

            @pl.when(u + 1 < nstrip)
            def _():
                idx_copy(u + 1, q).start()

        @pl.loop(0, nstrip, step=2)
        def _pair(u0):
            stage(u0, 0)
            stage(u0 + 1, 1)

        last = nstrip - 1
        gathers(last, 1, wait=True)
        tile_stores(last - 2, 1, wait=True)
        build_tiles(1)
        tile_stores(last, 1, wait=False)
        tile_stores(last - 1, 0, wait=True)
        tile_stores(last, 1, wait=True)

    return lookup


def kernel(indices, weight):
    NT, T = indices.shape
    out5 = _make_lookup(NT, weight.shape[0])(
        weight, indices.T.astype(jnp.int32)
    )
    return out5.transpose(2, 4, 0, 1, 3).reshape(NT, T, _D)


# speedup vs baseline: 1.8238x; 1.3748x over previous
import functools

import jax
import jax.numpy as jnp
from jax import lax
from jax.experimental import pallas as pl
from jax.experimental.pallas import tpu as pltpu
from jax.experimental.pallas import tpu_sc as plsc

_D = 32
_T = 50
_J = 5
_IB = 128
_DT = _D // 8


@functools.cache
def _make_lookup(NT: int, V: int):
    info = plsc.get_sparse_core_info()
    nc, ns = info.num_cores, info.num_subcores
    nw = nc * ns
    t_per_w = NT // nw
    nib = t_per_w // _IB
    nstrip = nib * (_T // _J)
    assert NT % nw == 0 and t_per_w % _IB == 0 and _T % _J == 0
    assert nstrip % 2 == 0

    mesh = plsc.VectorSubcoreMesh(core_axis_name="c", subcore_axis_name="s")

    @functools.partial(
        pl.kernel,
        mesh=mesh,
        compiler_params=pltpu.CompilerParams(
            use_tc_tiling_on_sc=False, needs_layout_passes=False
        ),
        out_type=jax.ShapeDtypeStruct((_T, _DT, NT // _IB, 8, _IB), jnp.float32),
        scratch_types=[
            pltpu.VMEM((_J, _IB), jnp.int32),
            pltpu.VMEM((_J, _IB), jnp.int32),
            pltpu.VMEM((_J, _IB, _D), jnp.float32),
            pltpu.VMEM((_J, _IB, _D), jnp.float32),
            pltpu.VMEM((_J, _DT, 8, _IB), jnp.float32),
            pltpu.VMEM((_J, _DT, 8, _IB), jnp.float32),
            pltpu.SemaphoreType.DMA,
            pltpu.SemaphoreType.DMA,
            pltpu.SemaphoreType.DMA,
            pltpu.SemaphoreType.DMA,
            pltpu.SemaphoreType.DMA,
            pltpu.SemaphoreType.DMA,
        ],
    )
    def lookup(table_hbm, idxT_hbm, out_hbm,
               idx0, idx1, rows0, rows1, tiles0, tiles1,
               si0, si1, sg0, sg1, so0, so1):
        wid = lax.axis_index("s") * nc + lax.axis_index("c")
        ib0 = wid * nib
        idx_b = (idx0, idx1)
        rows_b = (rows0, rows1)
        tiles_b = (tiles0, tiles1)
        si = (si0, si1)
        sg = (sg0, sg1)
        so = (so0, so1)
        nsj = _T // _J

        def strip_coords(u):
            ib = ib0 + u // nsj
            jj0 = (u % nsj) * _J
            return ib, jj0

        def idx_copy(u, p):
            ib, jj0 = strip_coords(u)
            i0 = pl.multiple_of(ib * _IB, 128)
            return pltpu.make_async_copy(
                idxT_hbm.at[pl.ds(jj0, _J), pl.ds(i0, _IB)], idx_b[p], si[p]
            )

        def gathers(u, p, wait):
            for j in range(_J):
                cp = pltpu.make_async_copy(
                    table_hbm.at[idx_b[p].at[j]], rows_b[p].at[j], sg[p]
                )
                if wait:
                    cp.wait()
                else:
                    cp.start()

        def tile_stores(u, p, wait):
            ib, jj0 = strip_coords(u)
            for j in range(_J):
                for dt in range(_DT):
                    cp = pltpu.make_async_copy(
                        tiles_b[p].at[j, dt], out_hbm.at[jj0 + j, dt, ib], so[p]
                    )
                    if wait:
                        cp.wait()
                    else:
                        cp.start()

        def build_tiles(p):
            lanes = lax.iota(jnp.int32, 16)
            zeros = jnp.zeros((16,), jnp.int32)

            @pl.loop(0, _J)
            def _j(j):
                jv = zeros + j
                for k in range(2):

                    @pl.loop(0, _IB // 16)
                    def _ig(ig):
                        iv = lanes + ig * 16
                        ts = [(lanes + g) & 15 for g in range(16)]
                        vals = [
                            plsc.load_gather(
                                rows_b[p], [jv, iv, ts[g] + 16 * k]
                            )
                            for g in range(16)
                        ]
                        for g in range(16):
                            t = ts[g]
                            plsc.store_scatter(
                                tiles_b[p],
                                [jv, (t >> 3) + 2 * k, t & 7, iv],
                                vals[g],
                            )

        idx_copy(0, 0).start()

        def stage(u, p):
            q = 1 - p
            idx_copy(u, p).wait()
            gathers(u, p, wait=False)

            @pl.when(u >= 1)
            def _():
                gathers(u - 1, q, wait=True)

                @pl.when(u >= 3)
                def _():
                    tile_stores(u - 3, q, wait=True)

                build_tiles(q)
                tile_stores(u - 1, q, wait=False)

            @pl.when(u + 1 < nstrip)
            def _():
                idx_copy(u + 1, q).start()

        @pl.loop(0, nstrip, step=2)
        def _pair(u0):
            stage(u0, 0)
            stage(u0 + 1, 1)

        last = nstrip - 1
        gathers(last, 1, wait=True)
        tile_stores(last - 2, 1, wait=True)
        build_tiles(1)
        tile_stores(last, 1, wait=False)
        tile_stores(last - 1, 0, wait=True)
        tile_stores(last, 1, wait=True)

    return lookup


def kernel(indices, weight):
    NT, T = indices.shape
    out5 = _make_lookup(NT, weight.shape[0])(
        weight, indices.T.astype(jnp.int32)
    )
    return out5.transpose(2, 4, 0, 1, 3).reshape(NT, T, _D)
